# Initial kernel scaffold; baseline (speedup 1.0000x reference)
import jax
import jax.numpy as jnp
from jax.experimental import pallas as pl


def kernel(x, input_weights, process_indices, output_weights, input_neurons, process_neurons, output_neurons):
    raise NotImplementedError("write your pallas kernel here")


# speedup vs baseline: 2.3334x; 2.3334x over previous
import jax
import jax.numpy as jnp
from jax.experimental import pallas as pl

D_MODEL = 1024
RANK = 128
N_INPUT = 8
N_PROCESS = 32
N_OUTPUT = 8
K = 4

TILE = 512


def _body(x_ref, wi_ref, idx_ref, wo_ref, win_ref, pn_ref, won_ref, out_ref):
    T = x_ref.shape[0]
    xb = x_ref[...]
    wi = wi_ref[...]

    P = jnp.dot(xb, win_ref[...], preferred_element_type=jnp.float32)
    P = P.reshape(T, N_INPUT, RANK)
    h = jnp.sum(P * wi[:, :, None], axis=1)

    pn = pn_ref[...]
    pn = pn * jax.lax.rsqrt(jnp.sum(pn * pn, axis=1, keepdims=True) + 1e-8)
    idx = idx_ref[...]
    iota = jax.lax.broadcasted_iota(jnp.int32, (T, N_PROCESS), 1)
    for i in range(K):
        oh = (idx[:, i : i + 1] == iota).astype(jnp.float32)
        v = jnp.dot(oh, pn, preferred_element_type=jnp.float32)
        h = h - 2.0 * v * jnp.sum(h * v, axis=1, keepdims=True)

    wo = wo_ref[...]
    hw = (wo[:, :, None] * h[:, None, :]).reshape(T, N_OUTPUT * RANK)
    out_ref[...] = jnp.dot(hw, won_ref[...], preferred_element_type=jnp.float32)


def kernel(x, input_weights, process_indices, output_weights,
           input_neurons, process_neurons, output_neurons):
    B, S, D = x.shape
    N = B * S
    xf = x.reshape(N, D)
    wif = input_weights.reshape(N, N_INPUT)
    idxf = process_indices.reshape(N, K).astype(jnp.int32)
    wof = output_weights.reshape(N, N_OUTPUT)
    win_t = jnp.transpose(input_neurons, (1, 0, 2)).reshape(D, N_INPUT * RANK)
    won_f = output_neurons.reshape(N_OUTPUT * RANK, D)

    grid = (N // TILE,)
    out = pl.pallas_call(
        _body,
        grid=grid,
        in_specs=[
            pl.BlockSpec((TILE, D), lambda i: (i, 0)),
            pl.BlockSpec((TILE, N_INPUT), lambda i: (i, 0)),
            pl.BlockSpec((TILE, K), lambda i: (i, 0)),
            pl.BlockSpec((TILE, N_OUTPUT), lambda i: (i, 0)),
            pl.BlockSpec((D, N_INPUT * RANK), lambda i: (0, 0)),
            pl.BlockSpec((N_PROCESS, RANK), lambda i: (0, 0)),
            pl.BlockSpec((N_OUTPUT * RANK, D), lambda i: (0, 0)),
        ],
        out_specs=pl.BlockSpec((TILE, D), lambda i: (i, 0)),
        out_shape=jax.ShapeDtypeStruct((N, D), jnp.float32),
    )(xf, wif, idxf, wof, win_t, process_neurons, won_f)
    return out.reshape(B, S, D)
